# baseline (speedup 1.0000x reference)
import functools

import jax
import jax.numpy as jnp
from jax import lax
from jax.experimental import pallas as pl
from jax.experimental.pallas import tpu as pltpu
from jax.experimental.pallas import tpu_sc as plsc

VOCAB = 1000000
DIM = 32
BATCH = 4096
AR_LEN = 200

NUM_WORKERS = 32
BBLK = 512
NB = BATCH // BBLK
ITEMS = AR_LEN * NB
IPT = ITEMS // NUM_WORKERS
NPAIR = IPT // 2

_mesh = plsc.VectorSubcoreMesh(core_axis_name="c", subcore_axis_name="s")


@functools.partial(
    pl.kernel,
    out_type=jax.ShapeDtypeStruct((AR_LEN, DIM, BATCH), jnp.float32),
    mesh=_mesh,
    scratch_types=[
        [pltpu.VMEM((BBLK,), jnp.int32) for _ in range(2)],
        [pltpu.VMEM((BBLK, DIM), jnp.float32) for _ in range(2)],
        pltpu.VMEM((DIM, BBLK + 1), jnp.float32),
        [pltpu.SemaphoreType.DMA for _ in range(2)],
        pltpu.SemaphoreType.DMA,
    ],
    compiler_params=pltpu.CompilerParams(
        use_tc_tiling_on_sc=False, needs_layout_passes=False
    ),
)
def _gather_t(ids_hbm, table_hbm, out_hbm, idxv, rows, slab_t, gsems, osem):
    w = lax.axis_index("s") * 2 + lax.axis_index("c")
    kbase = w * IPT
    iota = lax.iota(jnp.int32, 16)

    def stage_and_fire(k, p):
        item = kbase + k
        t = item // NB
        bb = item % NB
        pltpu.sync_copy(ids_hbm.at[pl.ds(t * BATCH + bb * BBLK, BBLK)], idxv[p])
        pltpu.async_copy(table_hbm.at[idxv[p]], rows[p], gsems[p])

    def wait_gather(p):
        pltpu.make_async_copy(table_hbm.at[pl.ds(0, BBLK)], rows[p], gsems[p]).wait()

    def drain_stores():
        pltpu.make_async_copy(
            out_hbm.at[0, pl.ds(0, DIM), pl.ds(0, BBLK)],
            slab_t.at[pl.ds(0, DIM), pl.ds(0, BBLK)],
            osem,
        ).wait()

    iota_hi = iota + 16

    def transpose_and_store(k, p):
        item = kbase + k
        t = item // NB
        bb = item % NB

        @plsc.parallel_loop(0, BBLK, step=1, unroll=8)
        def _(b):
            bvec = jnp.broadcast_to(b, (16,)).astype(jnp.int32)
            v0 = rows[p][b, pl.ds(0, 16)]
            v1 = rows[p][b, pl.ds(16, 16)]
            plsc.store_scatter(slab_t, [iota, bvec], v0)
            plsc.store_scatter(slab_t, [iota_hi, bvec], v1)

        for c in range(DIM):
            pltpu.async_copy(
                slab_t.at[c, pl.ds(0, BBLK)],
                out_hbm.at[t, c, pl.ds(bb * BBLK, BBLK)],
                osem,
            )

    stage_and_fire(0, 0)

    def pair(j, carry):
        k0 = 2 * j
        stage_and_fire(k0 + 1, 1)
        wait_gather(0)

        @pl.when(j > 0)
        def _():
            drain_stores()

        transpose_and_store(k0, 0)

        @pl.when(k0 + 2 < IPT)
        def _():
            stage_and_fire(k0 + 2, 0)

        wait_gather(1)
        drain_stores()
        transpose_and_store(k0 + 1, 1)
        return carry

    lax.fori_loop(0, NPAIR, pair, 0)
    drain_stores()


TBLK = 8192
_TGRID = -(-VOCAB // TBLK)


@functools.partial(
    pl.pallas_call,
    out_shape=jax.ShapeDtypeStruct((VOCAB * DIM // 128, 128), jnp.float32),
    grid=(_TGRID,),
    in_specs=[pl.BlockSpec((DIM, TBLK), lambda g: (0, g))],
    out_specs=pl.BlockSpec((TBLK * DIM // 128, 128), lambda g: (g, 0)),
)
def _retile(xt_ref, o_ref):
    x = xt_ref[...]
    o_ref[...] = jnp.transpose(x.reshape(DIM, TBLK // 4, 4), (1, 2, 0)).reshape(
        TBLK * DIM // 128, 128
    )


def kernel(input_ids, W):
    w_rm = _retile(W.T).reshape(VOCAB, DIM)
    ids_t = input_ids.T.reshape(-1)
    out_t = _gather_t(ids_t, w_rm)
    return jnp.transpose(out_t, (2, 0, 1))


# speedup vs baseline: 3.8297x; 3.8297x over previous
import functools

import jax
import jax.numpy as jnp
from jax import lax
from jax.experimental import pallas as pl
from jax.experimental.pallas import tpu as pltpu
from jax.experimental.pallas import tpu_sc as plsc

VOCAB = 1000000
DIM = 32
BATCH = 4096
AR_LEN = 200

NUM_WORKERS = 32
BBLK = 1024
NB = BATCH // BBLK
ITEMS = AR_LEN * NB
IPT = ITEMS // NUM_WORKERS
NPAIR = IPT // 2

_mesh = plsc.VectorSubcoreMesh(core_axis_name="c", subcore_axis_name="s")


@functools.partial(
    pl.kernel,
    out_type=jax.ShapeDtypeStruct((AR_LEN, DIM, BATCH), jnp.float32),
    mesh=_mesh,
    scratch_types=[
        [pltpu.VMEM((BBLK,), jnp.int32) for _ in range(2)],
        [pltpu.VMEM((BBLK, DIM), jnp.float32) for _ in range(2)],
        pltpu.VMEM((DIM, BBLK + 1), jnp.float32),
        [pltpu.SemaphoreType.DMA for _ in range(2)],
        pltpu.SemaphoreType.DMA,
    ],
    compiler_params=pltpu.CompilerParams(
        use_tc_tiling_on_sc=False, needs_layout_passes=False
    ),
)
def _gather_t(ids_hbm, table_hbm, out_hbm, idxv, rows, slab_t, gsems, osem):
    w = lax.axis_index("s") * 2 + lax.axis_index("c")
    kbase = w * IPT
    iota = lax.iota(jnp.int32, 16)

    def stage_and_fire(k, p):
        item = kbase + k
        t = item // NB
        bb = item % NB
        pltpu.sync_copy(ids_hbm.at[pl.ds(t * BATCH + bb * BBLK, BBLK)], idxv[p])
        pltpu.async_copy(table_hbm.at[idxv[p]], rows[p], gsems[p])

    def wait_gather(p):
        pltpu.make_async_copy(table_hbm.at[pl.ds(0, BBLK)], rows[p], gsems[p]).wait()

    def drain_stores():
        pltpu.make_async_copy(
            out_hbm.at[0, pl.ds(0, DIM), pl.ds(0, BBLK)],
            slab_t.at[pl.ds(0, DIM), pl.ds(0, BBLK)],
            osem,
        ).wait()

    iota_hi = iota + 16

    def transpose_and_store(k, p):
        item = kbase + k
        t = item // NB
        bb = item % NB

        @plsc.parallel_loop(0, BBLK, step=1, unroll=8)
        def _(b):
            bvec = jnp.broadcast_to(b, (16,)).astype(jnp.int32)
            v0 = rows[p][b, pl.ds(0, 16)]
            v1 = rows[p][b, pl.ds(16, 16)]
            plsc.store_scatter(slab_t, [iota, bvec], v0)
            plsc.store_scatter(slab_t, [iota_hi, bvec], v1)

        for c in range(DIM):
            pltpu.async_copy(
                slab_t.at[c, pl.ds(0, BBLK)],
                out_hbm.at[t, c, pl.ds(bb * BBLK, BBLK)],
                osem,
            )

    stage_and_fire(0, 0)

    def pair(j, carry):
        k0 = 2 * j
        stage_and_fire(k0 + 1, 1)
        wait_gather(0)

        @pl.when(j > 0)
        def _():
            drain_stores()

        transpose_and_store(k0, 0)

        @pl.when(k0 + 2 < IPT)
        def _():
            stage_and_fire(k0 + 2, 0)

        wait_gather(1)
        drain_stores()
        transpose_and_store(k0 + 1, 1)
        return carry

    lax.fori_loop(0, NPAIR, pair, 0)
    if IPT % 2 == 1:
        wait_gather(0)
        drain_stores()
        transpose_and_store(IPT - 1, 0)
    drain_stores()


def kernel(input_ids, W):
    ids_t = input_ids.T.reshape(-1)
    out_t = _gather_t(ids_t, W)
    return jnp.transpose(out_t, (2, 0, 1))
